# Initial kernel scaffold; baseline (speedup 1.0000x reference)
import jax
import jax.numpy as jnp
from jax.experimental import pallas as pl


def kernel(x, lengths, emb, W1, b1, W2, b2):
    raise NotImplementedError("write your pallas kernel here")


# speedup vs baseline: 10.1178x; 10.1178x over previous
import functools

import jax
import jax.numpy as jnp
from jax import lax
from jax.experimental import pallas as pl
from jax.experimental.pallas import tpu as pltpu
from jax.experimental.pallas import tpu_sc as plsc

B, L = 4096, 200
DIM = 64
DIM_PAD = 128
HIDDEN = 1000
OUT_PAD = 128
NC, NS = 2, 16
NW = NC * NS
SPT = B // NW
HALF = L // 2
LANES = 16
NCHUNK = DIM // LANES


def _sc_embed_sum(x_r, emb):
    mesh = plsc.VectorSubcoreMesh(
        core_axis_name="c", subcore_axis_name="s", num_cores=NC, num_subcores=NS
    )

    @functools.partial(
        pl.kernel,
        out_type=jax.ShapeDtypeStruct((B, DIM), jnp.float32),
        mesh=mesh,
        scratch_types=[
            pltpu.VMEM((SPT, 2, HALF), jnp.int32),
            pltpu.VMEM((2, L, DIM_PAD), jnp.float32),
            pltpu.VMEM((SPT, DIM), jnp.float32),
            pltpu.SemaphoreType.DMA,
            pltpu.SemaphoreType.DMA,
        ],
    )
    def k(x_hbm, emb_hbm, out_hbm, idx_v, rows_v, out_v, sem0, sem1):
        wid = lax.axis_index("s") * NC + lax.axis_index("c")
        base = wid * SPT
        pltpu.sync_copy(x_hbm.at[pl.ds(base, SPT)], idx_v)
        sems = (sem0, sem1)

        def issue(i, b):
            pltpu.async_copy(
                emb_hbm.at[idx_v.at[i, 0]], rows_v.at[b, pl.ds(0, HALF)], sems[b]
            )
            pltpu.async_copy(
                emb_hbm.at[idx_v.at[i, 1]], rows_v.at[b, pl.ds(HALF, HALF)], sems[b]
            )

        def wait(i, b):
            pltpu.make_async_copy(
                emb_hbm.at[idx_v.at[i, 0]], rows_v.at[b, pl.ds(0, HALF)], sems[b]
            ).wait()
            pltpu.make_async_copy(
                emb_hbm.at[idx_v.at[i, 1]], rows_v.at[b, pl.ds(HALF, HALF)], sems[b]
            ).wait()

        def accumulate(i, b):
            def body(j, carry):
                accs = list(carry)
                for r in range(8):
                    row = j * 8 + r
                    for d in range(NCHUNK):
                        accs[d] = accs[d] + rows_v[b, row, pl.ds(d * LANES, LANES)]
                return tuple(accs)

            zero = jnp.zeros((LANES,), jnp.float32)
            accs = lax.fori_loop(0, L // 8, body, (zero,) * NCHUNK)
            for d in range(NCHUNK):
                out_v[i, pl.ds(d * LANES, LANES)] = accs[d]

        issue(0, 0)
        issue(1, 1)

        def outer(o, carry):
            i0 = o * 2
            wait(i0, 0)
            accumulate(i0, 0)
            issue(i0 + 2, 0)
            wait(i0 + 1, 1)
            accumulate(i0 + 1, 1)
            issue(i0 + 3, 1)
            return carry

        lax.fori_loop(0, SPT // 2 - 1, outer, 0)
        wait(SPT - 2, 0)
        accumulate(SPT - 2, 0)
        wait(SPT - 1, 1)
        accumulate(SPT - 1, 1)

        pltpu.sync_copy(out_v, out_hbm.at[pl.ds(base, SPT)])

    return k(x_r, emb)


def _tc_mlp(rep_sum, len_f, W1, b1r, W2p, b2p):
    BLK = 512

    def body(rep_ref, len_ref, w1_ref, b1_ref, w2_ref, b2_ref, out_ref):
        rep = rep_ref[...] / len_ref[...]
        h = jnp.dot(rep, w1_ref[...], preferred_element_type=jnp.float32)
        h = jnp.maximum(h + b1_ref[...], 0.0)
        out_ref[...] = (
            jnp.dot(h, w2_ref[...], preferred_element_type=jnp.float32) + b2_ref[...]
        )

    return pl.pallas_call(
        body,
        grid=(B // BLK,),
        in_specs=[
            pl.BlockSpec((BLK, DIM), lambda i: (i, 0)),
            pl.BlockSpec((BLK, 1), lambda i: (i, 0)),
            pl.BlockSpec((DIM, HIDDEN), lambda i: (0, 0)),
            pl.BlockSpec((1, HIDDEN), lambda i: (0, 0)),
            pl.BlockSpec((HIDDEN, OUT_PAD), lambda i: (0, 0)),
            pl.BlockSpec((1, OUT_PAD), lambda i: (0, 0)),
        ],
        out_specs=pl.BlockSpec((BLK, OUT_PAD), lambda i: (i, 0)),
        out_shape=jax.ShapeDtypeStruct((B, OUT_PAD), jnp.float32),
    )(rep_sum, len_f, W1, b1r, W2p, b2p)


def kernel(x, lengths, emb, W1, b1, W2, b2):
    x_r = x.astype(jnp.int32).reshape(B, 2, HALF)
    emb_p = jnp.pad(emb, ((0, 0), (0, DIM_PAD - DIM)))
    rep_sum = _sc_embed_sum(x_r, emb_p)
    len_f = lengths.astype(jnp.float32).reshape(B, 1)
    b1r = b1.reshape(1, HIDDEN)
    W2p = jnp.pad(W2, ((0, 0), (0, OUT_PAD - W2.shape[1])))
    b2p = jnp.pad(b2, (0, OUT_PAD - b2.shape[0])).reshape(1, OUT_PAD)
    logits_pad = _tc_mlp(rep_sum, len_f, W1, b1r, W2p, b2p)
    return logits_pad[:, : W2.shape[1]]
